# baseline (speedup 1.0000x reference)
import functools

import jax
import jax.numpy as jnp
from jax import lax
from jax.experimental import pallas as pl
from jax.experimental.pallas import tpu as pltpu
from jax.experimental.pallas import tpu_sc as plsc

_B = 4096
_V = 2048
_NOBJ = 10
_L = 16
_NC = 2
_NS = 16
_ROW = 3 * _V
_CHUNK = 2
_EPB = _B // _NS


def _param_body(a_ref, g0_ref, g1_ref, g2_ref, t0_ref, t1_ref, t2_ref, out_ref):
    a = a_ref[...]
    na = jnp.abs(a)
    small_a = na < 1e-6
    safe_a = jnp.where(small_a, jnp.ones_like(na), na)
    sa = jnp.where(small_a, 0.5 - na * na / 48.0, jnp.sin(0.5 * na) / safe_a)
    aw = jnp.cos(0.5 * na)
    az = -a * sa

    g0 = g0_ref[...]
    g1 = g1_ref[...]
    g2 = g2_ref[...]
    n2 = g0 * g0 + g1 * g1 + g2 * g2
    ng = jnp.sqrt(n2)
    small_g = ng < 1e-6
    safe_g = jnp.where(small_g, jnp.ones_like(ng), ng)
    sg = jnp.where(small_g, 0.5 - n2 / 48.0, jnp.sin(0.5 * ng) / safe_g)
    gw = jnp.cos(0.5 * ng)
    gx = g0 * sg
    gy = g1 * sg
    gz = g2 * sg

    tw = gw * aw - gz * az
    tx = gx * aw + gy * az
    ty = gy * aw - gx * az
    tz = gw * az + gz * aw

    def rot(w, x, y, z):
        return (1 - 2 * (y * y + z * z), 2 * (x * y - w * z), 2 * (x * z + w * y),
                2 * (x * y + w * z), 1 - 2 * (x * x + z * z), 2 * (y * z - w * x),
                2 * (x * z - w * y), 2 * (y * z + w * x), 1 - 2 * (x * x + y * y))

    vals = list(rot(gw, gx, gy, gz)) + list(rot(tw, tx, ty, tz))
    vals += [t0_ref[...], t1_ref[...], t2_ref[...]]
    for k, v in enumerate(vals):
        out_ref[k, :, :] = v


def _sc_body(tabs_hbm, idx_hbm, par_hbm, bv_hbm, tv_hbm,
             tab_v, idx_v, par_v, obuf_v):
    cid = lax.axis_index("c")
    sid = lax.axis_index("s")
    base = sid * _EPB

    pltpu.sync_copy(tabs_hbm.at[cid], tab_v)
    pltpu.sync_copy(idx_hbm.at[pl.ds(base, _EPB)], idx_v)
    pltpu.sync_copy(par_hbm.at[:, pl.ds(base, _EPB)], par_v)

    iota3 = 3 * lax.iota(jnp.int32, _L)

    def full(v):
        return jnp.full((_L,), v, jnp.int32)

    rbase = cid * 9

    def process_element(i_loc, e_off):
        o = jnp.max(plsc.load_gather(idx_v, [full(i_loc)]))
        col = full(i_loc)
        R = [plsc.load_gather(par_v, [full(rbase + m), col]) for m in range(9)]
        T = [plsc.load_gather(par_v, [full(18 + m), col]) for m in range(3)]

        def gbody(g, carry):
            v0 = g * _L
            xs = tab_v[o, 0, pl.ds(v0, _L)]
            ys = tab_v[o, 1, pl.ds(v0, _L)]
            zs = tab_v[o, 2, pl.ds(v0, _L)]
            off = e_off + 48 * g
            for c in range(3):
                val = R[3 * c] * xs + R[3 * c + 1] * ys + R[3 * c + 2] * zs + T[c]
                plsc.store_scatter(obuf_v, [iota3 + (off + c)], val)
            return carry

        lax.fori_loop(0, _V // _L, gbody, 0)

    def chunk_body(j, carry):
        i0 = _CHUNK * j
        for e in range(_CHUNK):
            process_element(i0 + e, e * _ROW)
        dst = pl.ds((base + i0) * _ROW, _CHUNK * _ROW)

        @pl.when(cid == 0)
        def _():
            pltpu.sync_copy(obuf_v, bv_hbm.at[dst])

        @pl.when(cid == 1)
        def _():
            pltpu.sync_copy(obuf_v, tv_hbm.at[dst])

        return carry

    lax.fori_loop(0, _EPB // _CHUNK, chunk_body, 0)


@jax.jit
def kernel(angles, global_orient, transl, obj_idx, bottom_v, top_v):
    a_r = angles.reshape(32, 128)
    gt = global_orient.T
    tt = transl.T
    g0, g1, g2 = (gt[i].reshape(32, 128) for i in range(3))
    t0, t1, t2 = (tt[i].reshape(32, 128) for i in range(3))

    params3 = pl.pallas_call(
        _param_body,
        out_shape=jax.ShapeDtypeStruct((21, 32, 128), jnp.float32),
    )(a_r, g0, g1, g2, t0, t1, t2)
    params = params3.reshape(21, _B)

    tabs = jnp.stack([bottom_v.transpose(0, 2, 1), top_v.transpose(0, 2, 1)])
    idx = obj_idx.astype(jnp.int32)

    mesh = plsc.VectorSubcoreMesh(
        core_axis_name="c", subcore_axis_name="s",
        num_cores=_NC, num_subcores=_NS)

    run = pl.kernel(
        _sc_body,
        out_type=(
            jax.ShapeDtypeStruct((_B * _ROW,), jnp.float32),
            jax.ShapeDtypeStruct((_B * _ROW,), jnp.float32),
        ),
        mesh=mesh,
        compiler_params=pltpu.CompilerParams(needs_layout_passes=False),
        scratch_types=[
            pltpu.VMEM((_NOBJ, 3, _V), jnp.float32),
            pltpu.VMEM((_EPB,), jnp.int32),
            pltpu.VMEM((21, _EPB), jnp.float32),
            pltpu.VMEM((_CHUNK * _ROW,), jnp.float32),
        ],
    )

    bvf, tvf = run(tabs, idx, params)
    return bvf.reshape(_B, _V, 3), tvf.reshape(_B, _V, 3)


# speedup vs baseline: 1.0259x; 1.0259x over previous
import functools

import jax
import jax.numpy as jnp
from jax import lax
from jax.experimental import pallas as pl
from jax.experimental.pallas import tpu as pltpu
from jax.experimental.pallas import tpu_sc as plsc

_B = 4096
_V = 2048
_NOBJ = 10
_L = 16
_NC = 2
_NS = 16
_NW = _NC * _NS
_ROW = 3 * _V
_CHUNK = 2
_EPB = _B // _NW


def _param_body(a_ref, g0_ref, g1_ref, g2_ref, t0_ref, t1_ref, t2_ref, out_ref):
    a = a_ref[...]
    na = jnp.abs(a)
    small_a = na < 1e-6
    safe_a = jnp.where(small_a, jnp.ones_like(na), na)
    sa = jnp.where(small_a, 0.5 - na * na / 48.0, jnp.sin(0.5 * na) / safe_a)
    aw = jnp.cos(0.5 * na)
    az = -a * sa

    g0 = g0_ref[...]
    g1 = g1_ref[...]
    g2 = g2_ref[...]
    n2 = g0 * g0 + g1 * g1 + g2 * g2
    ng = jnp.sqrt(n2)
    small_g = ng < 1e-6
    safe_g = jnp.where(small_g, jnp.ones_like(ng), ng)
    sg = jnp.where(small_g, 0.5 - n2 / 48.0, jnp.sin(0.5 * ng) / safe_g)
    gw = jnp.cos(0.5 * ng)
    gx = g0 * sg
    gy = g1 * sg
    gz = g2 * sg

    tw = gw * aw - gz * az
    tx = gx * aw + gy * az
    ty = gy * aw - gx * az
    tz = gw * az + gz * aw

    def rot(w, x, y, z):
        return (1 - 2 * (y * y + z * z), 2 * (x * y - w * z), 2 * (x * z + w * y),
                2 * (x * y + w * z), 1 - 2 * (x * x + z * z), 2 * (y * z - w * x),
                2 * (x * z - w * y), 2 * (y * z + w * x), 1 - 2 * (x * x + y * y))

    vals = list(rot(gw, gx, gy, gz)) + list(rot(tw, tx, ty, tz))
    vals += [t0_ref[...], t1_ref[...], t2_ref[...]]
    for k, v in enumerate(vals):
        out_ref[k, :, :] = v


def _sc_body(rbase, tab_hbm, idx_hbm, par_hbm, out_hbm,
             tab_v, idx_v, par_v, obuf0_v, obuf1_v, sem0, sem1):
    cid = lax.axis_index("c")
    sid = lax.axis_index("s")
    wid = sid * _NC + cid
    base = wid * _EPB

    pltpu.sync_copy(tab_hbm, tab_v)
    pltpu.sync_copy(idx_hbm.at[pl.ds(base, _EPB)], idx_v)
    pltpu.sync_copy(par_hbm.at[:, pl.ds(base, _EPB)], par_v)

    iota3 = 3 * lax.iota(jnp.int32, _L)

    def full(v):
        return jnp.full((_L,), v, jnp.int32)

    def process_element(obuf, i_loc, e_off):
        o = jnp.max(plsc.load_gather(idx_v, [full(i_loc)]))
        col = full(i_loc)
        R = [plsc.load_gather(par_v, [full(rbase + m), col]) for m in range(9)]
        T = [plsc.load_gather(par_v, [full(18 + m), col]) for m in range(3)]

        def gbody(g, carry):
            v0 = g * _L
            xs = tab_v[o, 0, pl.ds(v0, _L)]
            ys = tab_v[o, 1, pl.ds(v0, _L)]
            zs = tab_v[o, 2, pl.ds(v0, _L)]
            off = e_off + 48 * g
            for c in range(3):
                val = (R[3 * c] * xs + R[3 * c + 1] * ys
                       + R[3 * c + 2] * zs + T[c])
                plsc.store_scatter(obuf, [iota3 + (off + c)], val)
            return carry

        lax.fori_loop(0, _V // _L, gbody, 0)

    def drain(obuf, sem):
        pltpu.make_async_copy(
            obuf, out_hbm.at[pl.ds(0, _CHUNK * _ROW)], sem).wait()

    def super_body(m, carry):
        for slot, obuf, sem in ((0, obuf0_v, sem0), (1, obuf1_v, sem1)):
            j = 2 * m + slot
            i0 = _CHUNK * j

            @pl.when(m >= 1)
            def _():
                drain(obuf, sem)

            for e in range(_CHUNK):
                process_element(obuf, i0 + e, e * _ROW)
            dst = pl.ds((base + i0) * _ROW, _CHUNK * _ROW)
            pltpu.async_copy(obuf, out_hbm.at[dst], sem)

        return carry

    lax.fori_loop(0, _EPB // (2 * _CHUNK), super_body, 0)
    drain(obuf0_v, sem0)
    drain(obuf1_v, sem1)


@jax.jit
def kernel(angles, global_orient, transl, obj_idx, bottom_v, top_v):
    a_r = angles.reshape(32, 128)
    gt = global_orient.T
    tt = transl.T
    g0, g1, g2 = (gt[i].reshape(32, 128) for i in range(3))
    t0, t1, t2 = (tt[i].reshape(32, 128) for i in range(3))

    params3 = pl.pallas_call(
        _param_body,
        out_shape=jax.ShapeDtypeStruct((21, 32, 128), jnp.float32),
    )(a_r, g0, g1, g2, t0, t1, t2)
    params = params3.reshape(21, _B)

    idx = obj_idx.astype(jnp.int32)

    mesh = plsc.VectorSubcoreMesh(
        core_axis_name="c", subcore_axis_name="s",
        num_cores=_NC, num_subcores=_NS)

    def make_run(rbase):
        return pl.kernel(
            functools.partial(_sc_body, rbase),
            out_type=jax.ShapeDtypeStruct((_B * _ROW,), jnp.float32),
            mesh=mesh,
            compiler_params=pltpu.CompilerParams(needs_layout_passes=False),
            scratch_types=[
                pltpu.VMEM((_NOBJ, 3, _V), jnp.float32),
                pltpu.VMEM((_EPB,), jnp.int32),
                pltpu.VMEM((21, _EPB), jnp.float32),
                pltpu.VMEM((_CHUNK * _ROW,), jnp.float32),
                pltpu.VMEM((_CHUNK * _ROW,), jnp.float32),
                pltpu.SemaphoreType.DMA,
                pltpu.SemaphoreType.DMA,
            ],
        )

    bvf = make_run(0)(bottom_v.transpose(0, 2, 1), idx, params)
    tvf = make_run(9)(top_v.transpose(0, 2, 1), idx, params)
    return bvf.reshape(_B, _V, 3), tvf.reshape(_B, _V, 3)
